# baseline (speedup 1.0000x reference)
import functools

import jax
import jax.numpy as jnp
from jax import lax
from jax.experimental import pallas as pl
from jax.experimental.pallas import tpu as pltpu
from jax.experimental.pallas import tpu_sc as plsc

B = 128
D = 128
NS = 16
RPW = B // NS
NCH = D // 16
MARGIN_ = 1.0

_mesh = plsc.VectorSubcoreMesh(core_axis_name="c", subcore_axis_name="s",
                               num_cores=1)


@functools.partial(
    pl.kernel,
    out_type=jax.ShapeDtypeStruct((16,), jnp.float32),
    mesh=_mesh,
    compiler_params=pltpu.CompilerParams(needs_layout_passes=False,
                                         use_tc_tiling_on_sc=False),
    scratch_types=[
        pltpu.VMEM((B,), jnp.int32),
        pltpu.VMEM((B,), jnp.int32),
        pltpu.VMEM((B,), jnp.int32),
        pltpu.VMEM((B,), jnp.int32),
        pltpu.VMEM((B,), jnp.int32),
        pltpu.VMEM((3 * RPW, D), jnp.float32),
        pltpu.VMEM((2 * RPW, D), jnp.float32),
        pltpu.VMEM((32,), jnp.int32),
        pltpu.VMEM((16,), jnp.int32),
        pltpu.VMEM((RPW * NCH, 16), jnp.float32),
        pltpu.VMEM((RPW * NCH, 16), jnp.float32),
        pltpu.VMEM((B, 16), jnp.float32),
        pltpu.VMEM((B, 16), jnp.float32),
        pltpu.VMEM((B,), jnp.int32),
        pltpu.VMEM((B,), jnp.int32),
        pltpu.VMEM((RPW * NCH,), jnp.int32),
        pltpu.VMEM((RPW * NCH,), jnp.int32),
        pltpu.VMEM((2, 16), jnp.float32),
        pltpu.VMEM((NS, 2, 16), jnp.float32),
        pltpu.VMEM((16,), jnp.float32),
        pltpu.VMEM_SHARED((NS, 2, 16), jnp.float32),
        pltpu.SemaphoreType.DMA,
    ],
)
def _transh_sc(ph_hbm, prl_hbm, pt_hbm, nrl_hbm, nt_hbm,
               ent_tab, rel_tab, nrm_sub, out_hbm,
               idx_phf, idx_ptf, idx_ntf, idx_prf, idx_nrf,
               ent_rows, rel_rows, ent_idx, rel_idx,
               pnr, nnr, pnc, nnc, sub_p, sub_n,
               rsub_p, rsub_n, part_v, all_v, out_v, shared, sem):
    c = lax.axis_index("c")
    s = lax.axis_index("s")
    base = s * RPW
    blk = s // 2
    i_off = (s % 2) * RPW

    icps = [
        pltpu.async_copy(ph_hbm, idx_phf, sem),
        pltpu.async_copy(pt_hbm, idx_ptf, sem),
        pltpu.async_copy(nt_hbm, idx_ntf, sem),
        pltpu.async_copy(prl_hbm, idx_prf, sem),
        pltpu.async_copy(nrl_hbm, idx_nrf, sem),
    ]
    for cp in icps:
        cp.wait()

    iota16 = lax.iota(jnp.int32, 16)
    for ch in range(NCH):
        sub_p[pl.ds(ch * 16, 16)] = idx_prf[pl.ds(ch * 16, 16)] * NCH + blk
        sub_n[pl.ds(ch * 16, 16)] = idx_nrf[pl.ds(ch * 16, 16)] * NCH + blk
    for q in range(RPW * NCH // 16):
        r_vec = base + (q * 16 + iota16) // NCH
        k_vec = (q * 16 + iota16) % NCH
        rsub_p[pl.ds(q * 16, 16)] = plsc.load_gather(idx_prf, [r_vec]) * NCH + k_vec
        rsub_n[pl.ds(q * 16, 16)] = plsc.load_gather(idx_nrf, [r_vec]) * NCH + k_vec

    m8 = base + iota16 % RPW
    lo8 = iota16 < RPW
    ent_idx[pl.ds(0, 16)] = jnp.where(lo8, plsc.load_gather(idx_phf, [m8]),
                                      plsc.load_gather(idx_ptf, [m8]))
    ent_idx[pl.ds(16, 16)] = plsc.load_gather(idx_ntf, [m8])
    rel_idx[...] = jnp.where(lo8, plsc.load_gather(idx_prf, [m8]),
                             plsc.load_gather(idx_nrf, [m8]))

    cps = [
        pltpu.async_copy(nrm_sub.at[sub_p], pnc, sem),
        pltpu.async_copy(nrm_sub.at[sub_n], nnc, sem),
        pltpu.async_copy(nrm_sub.at[rsub_p], pnr, sem),
        pltpu.async_copy(nrm_sub.at[rsub_n], nnr, sem),
        pltpu.async_copy(ent_tab.at[ent_idx.at[pl.ds(0, 3 * RPW)]], ent_rows, sem),
        pltpu.async_copy(rel_tab.at[rel_idx], rel_rows, sem),
    ]
    for cp in cps:
        cp.wait()

    zero = jnp.zeros((16,), jnp.float32)
    accp = zero
    accn = zero
    for i_loc in range(RPW):
        i_vec = jnp.full((16,), i_off + i_loc, jnp.int32)
        for ch in range(NCH):
            col = ch * 16
            j_vec = col + iota16
            eh_v = ent_rows[i_loc, pl.ds(col, 16)]
            et_v = ent_rows[RPW + i_loc, pl.ds(col, 16)]
            ent_v = ent_rows[2 * RPW + i_loc, pl.ds(col, 16)]
            pr_v = rel_rows[i_loc, pl.ds(col, 16)]
            nr_v = rel_rows[RPW + i_loc, pl.ds(col, 16)]
            pn_row = pnr[i_loc * NCH + ch, :]
            nn_row = nnr[i_loc * NCH + ch, :]
            pnT = plsc.load_gather(pnc, [j_vec, i_vec])
            nnT = plsc.load_gather(nnc, [j_vec, i_vec])
            fp = 1.0 - pnT * pn_row
            fn = 1.0 - nnT * nn_row
            a_pos = (eh_v - et_v) * fp + pr_v
            a_neg = eh_v * fp - ent_v * fn + nr_v
            accp = accp + jnp.abs(a_pos)
            accn = accn + jnp.abs(a_neg)

    part_v[0, :] = accp
    part_v[1, :] = accn
    pltpu.sync_copy(part_v, shared.at[s])

    plsc.subcore_barrier()

    @pl.when(jnp.logical_and(c == 0, s == 0))
    def _():
        pltpu.sync_copy(shared, all_v)
        sp = zero
        sn = zero
        for w in range(NS):
            sp = sp + all_v[w, 0, :]
            sn = sn + all_v[w, 1, :]
        s_pos = jnp.sum(sp)
        s_neg = jnp.sum(sn)
        loss = jnp.maximum(0.0, s_neg * s_neg - s_pos * s_pos + MARGIN_)
        out_v[...] = jnp.full((16,), loss, jnp.float32)
        pltpu.sync_copy(out_v, out_hbm)


def kernel(pos_head, pos_rel, pos_tail, neg_head, neg_rel, neg_tail,
           entity_table, relation_table, normal_table):
    del neg_head
    out = _transh_sc(
        pos_head.astype(jnp.int32),
        pos_rel.astype(jnp.int32),
        pos_tail.astype(jnp.int32),
        neg_rel.astype(jnp.int32),
        neg_tail.astype(jnp.int32),
        entity_table,
        relation_table,
        normal_table.reshape(-1, 16),
    )
    return out[0]


# speedup vs baseline: 1.0101x; 1.0101x over previous
import functools

import jax
import jax.numpy as jnp
from jax import lax
from jax.experimental import pallas as pl
from jax.experimental.pallas import tpu as pltpu
from jax.experimental.pallas import tpu_sc as plsc

B = 128
D = 128
NS = 16
RPW = B // NS
NCH = D // 16
MARGIN_ = 1.0

_mesh = plsc.VectorSubcoreMesh(core_axis_name="c", subcore_axis_name="s",
                               num_cores=1)


@functools.partial(
    pl.kernel,
    out_type=jax.ShapeDtypeStruct((16,), jnp.float32),
    mesh=_mesh,
    compiler_params=pltpu.CompilerParams(needs_layout_passes=False,
                                         use_tc_tiling_on_sc=False,
                                         disable_bounds_checks=True,
                                         disable_semaphore_checks=True),
    scratch_types=[
        pltpu.VMEM((B,), jnp.int32),
        pltpu.VMEM((B,), jnp.int32),
        pltpu.VMEM((B,), jnp.int32),
        pltpu.VMEM((B,), jnp.int32),
        pltpu.VMEM((B,), jnp.int32),
        pltpu.VMEM((3 * RPW, D), jnp.float32),
        pltpu.VMEM((2 * RPW, D), jnp.float32),
        pltpu.VMEM((32,), jnp.int32),
        pltpu.VMEM((16,), jnp.int32),
        pltpu.VMEM((RPW * NCH, 16), jnp.float32),
        pltpu.VMEM((RPW * NCH, 16), jnp.float32),
        pltpu.VMEM((B, 16), jnp.float32),
        pltpu.VMEM((B, 16), jnp.float32),
        pltpu.VMEM((B,), jnp.int32),
        pltpu.VMEM((B,), jnp.int32),
        pltpu.VMEM((RPW * NCH,), jnp.int32),
        pltpu.VMEM((RPW * NCH,), jnp.int32),
        pltpu.VMEM((2, 16), jnp.float32),
        pltpu.VMEM((NS, 2, 16), jnp.float32),
        pltpu.VMEM((16,), jnp.float32),
        pltpu.VMEM_SHARED((NS, 2, 16), jnp.float32),
        pltpu.SemaphoreType.DMA,
    ],
)
def _transh_sc(ph_hbm, prl_hbm, pt_hbm, nrl_hbm, nt_hbm,
               ent_tab, rel_tab, nrm_sub, out_hbm,
               idx_phf, idx_ptf, idx_ntf, idx_prf, idx_nrf,
               ent_rows, rel_rows, ent_idx, rel_idx,
               pnr, nnr, pnc, nnc, sub_p, sub_n,
               rsub_p, rsub_n, part_v, all_v, out_v, shared, sem):
    c = lax.axis_index("c")
    s = lax.axis_index("s")
    base = s * RPW
    blk = s // 2
    i_off = (s % 2) * RPW

    icps = [
        pltpu.async_copy(ph_hbm, idx_phf, sem),
        pltpu.async_copy(pt_hbm, idx_ptf, sem),
        pltpu.async_copy(nt_hbm, idx_ntf, sem),
        pltpu.async_copy(prl_hbm, idx_prf, sem),
        pltpu.async_copy(nrl_hbm, idx_nrf, sem),
    ]
    for cp in icps:
        cp.wait()

    iota16 = lax.iota(jnp.int32, 16)
    m8 = base + iota16 % RPW
    lo8 = iota16 < RPW
    ent_idx[pl.ds(0, 16)] = jnp.where(lo8, plsc.load_gather(idx_phf, [m8]),
                                      plsc.load_gather(idx_ptf, [m8]))
    ent_idx[pl.ds(16, 16)] = plsc.load_gather(idx_ntf, [m8])
    rel_idx[...] = jnp.where(lo8, plsc.load_gather(idx_prf, [m8]),
                             plsc.load_gather(idx_nrf, [m8]))
    cps = [
        pltpu.async_copy(ent_tab.at[ent_idx.at[pl.ds(0, 3 * RPW)]], ent_rows, sem),
        pltpu.async_copy(rel_tab.at[rel_idx], rel_rows, sem),
    ]

    for ch in range(NCH):
        sub_p[pl.ds(ch * 16, 16)] = idx_prf[pl.ds(ch * 16, 16)] * NCH + blk
        sub_n[pl.ds(ch * 16, 16)] = idx_nrf[pl.ds(ch * 16, 16)] * NCH + blk
    for q in range(RPW * NCH // 16):
        r_vec = base + (q * 16 + iota16) // NCH
        k_vec = (q * 16 + iota16) % NCH
        rsub_p[pl.ds(q * 16, 16)] = plsc.load_gather(idx_prf, [r_vec]) * NCH + k_vec
        rsub_n[pl.ds(q * 16, 16)] = plsc.load_gather(idx_nrf, [r_vec]) * NCH + k_vec

    cps += [
        pltpu.async_copy(nrm_sub.at[sub_p], pnc, sem),
        pltpu.async_copy(nrm_sub.at[sub_n], nnc, sem),
        pltpu.async_copy(nrm_sub.at[rsub_p], pnr, sem),
        pltpu.async_copy(nrm_sub.at[rsub_n], nnr, sem),
    ]
    for cp in cps:
        cp.wait()

    zero = jnp.zeros((16,), jnp.float32)
    accp = zero
    accn = zero
    for i_loc in range(RPW):
        i_vec = jnp.full((16,), i_off + i_loc, jnp.int32)
        for ch in range(NCH):
            col = ch * 16
            j_vec = col + iota16
            eh_v = ent_rows[i_loc, pl.ds(col, 16)]
            et_v = ent_rows[RPW + i_loc, pl.ds(col, 16)]
            ent_v = ent_rows[2 * RPW + i_loc, pl.ds(col, 16)]
            pr_v = rel_rows[i_loc, pl.ds(col, 16)]
            nr_v = rel_rows[RPW + i_loc, pl.ds(col, 16)]
            pn_row = pnr[i_loc * NCH + ch, :]
            nn_row = nnr[i_loc * NCH + ch, :]
            pnT = plsc.load_gather(pnc, [j_vec, i_vec])
            nnT = plsc.load_gather(nnc, [j_vec, i_vec])
            fp = 1.0 - pnT * pn_row
            fn = 1.0 - nnT * nn_row
            a_pos = (eh_v - et_v) * fp + pr_v
            a_neg = eh_v * fp - ent_v * fn + nr_v
            accp = accp + jnp.abs(a_pos)
            accn = accn + jnp.abs(a_neg)

    part_v[0, :] = accp
    part_v[1, :] = accn
    pltpu.sync_copy(part_v, shared.at[s])

    plsc.subcore_barrier()

    @pl.when(jnp.logical_and(c == 0, s == 0))
    def _():
        pltpu.sync_copy(shared, all_v)
        sp = zero
        sn = zero
        for w in range(NS):
            sp = sp + all_v[w, 0, :]
            sn = sn + all_v[w, 1, :]
        s_pos = jnp.sum(sp)
        s_neg = jnp.sum(sn)
        loss = jnp.maximum(0.0, s_neg * s_neg - s_pos * s_pos + MARGIN_)
        out_v[...] = jnp.full((16,), loss, jnp.float32)
        pltpu.sync_copy(out_v, out_hbm)


def kernel(pos_head, pos_rel, pos_tail, neg_head, neg_rel, neg_tail,
           entity_table, relation_table, normal_table):
    del neg_head
    out = _transh_sc(
        pos_head.astype(jnp.int32),
        pos_rel.astype(jnp.int32),
        pos_tail.astype(jnp.int32),
        neg_rel.astype(jnp.int32),
        neg_tail.astype(jnp.int32),
        entity_table,
        relation_table,
        normal_table.reshape(-1, 16),
    )
    return out[0]
